# baseline (speedup 1.0000x reference)
import jax
import jax.numpy as jnp
from jax import lax
from jax.experimental import pallas as pl
from jax.experimental.pallas import tpu as pltpu
from jax.experimental.pallas import tpu_sc as plsc

NC = 2
NS = 16
NW = NC * NS

CHUNK = 128
NF = 32
BLK = 65536


def _dense_body(wt_ref, ht_ref, wb_ref, hb_ref, out_ref):
    prod = wt_ref[...] * ht_ref[...]
    out_ref[...] = jnp.sum(prod, axis=0) + wb_ref[...] + hb_ref[...]


def _gather_body(nodes_hbm, dsum_hbm, out_hbm, idx_v, val_v, sem):
    n_chunks = nodes_hbm.shape[0] // NW
    wid = lax.axis_index("s") * NC + lax.axis_index("c")
    base_row = wid * n_chunks

    pltpu.sync_copy(nodes_hbm.at[pl.ds(base_row, n_chunks)], idx_v)
    for k in range(n_chunks):
        pltpu.async_copy(dsum_hbm.at[idx_v.at[k]], val_v.at[k], sem)
    for k in range(n_chunks):
        pltpu.make_async_copy(dsum_hbm.at[idx_v.at[k]], val_v.at[k],
                              sem).wait()
    for k in range(n_chunks):
        pltpu.sync_copy(
            val_v.at[k],
            out_hbm.at[pl.ds((base_row + k) * CHUNK, CHUNK)])


def kernel(nodes, W, H, w_bias, h_bias):
    batch = nodes.shape[0]
    nn = W.shape[0]
    wt = jnp.transpose(W)
    ht = jnp.transpose(H)
    wb = jnp.reshape(w_bias, (nn,))
    hb = jnp.reshape(h_bias, (nn,))

    grid = (nn + BLK - 1) // BLK
    dsum = pl.pallas_call(
        _dense_body,
        grid=(grid,),
        in_specs=[
            pl.BlockSpec((NF, BLK), lambda i: (0, i)),
            pl.BlockSpec((NF, BLK), lambda i: (0, i)),
            pl.BlockSpec((BLK,), lambda i: (i,)),
            pl.BlockSpec((BLK,), lambda i: (i,)),
        ],
        out_specs=pl.BlockSpec((BLK,), lambda i: (i,)),
        out_shape=jax.ShapeDtypeStruct((nn,), jnp.float32),
        compiler_params=pltpu.CompilerParams(
            dimension_semantics=("arbitrary",)),
    )(wt, ht, wb, hb)

    nodes2d = jnp.reshape(nodes.astype(jnp.int32), (batch // CHUNK, CHUNK))
    mesh = plsc.VectorSubcoreMesh(core_axis_name="c", subcore_axis_name="s")
    n_chunks = nodes2d.shape[0] // NW
    run = pl.kernel(
        _gather_body,
        out_type=jax.ShapeDtypeStruct((batch,), jnp.float32),
        mesh=mesh,
        scratch_types=[
            pltpu.VMEM((n_chunks, CHUNK), jnp.int32),
            pltpu.VMEM((n_chunks, CHUNK), jnp.float32),
            pltpu.SemaphoreType.DMA,
        ],
    )
    return run(nodes2d, dsum)
